# baseline (speedup 1.0000x reference)
import functools

import jax
import jax.numpy as jnp
from jax.experimental import pallas as pl
from jax.experimental.pallas import tpu as pltpu
from jax.experimental.pallas import tpu_sc as plsc

_N = 64
_V = 100000
_T = 512
_H = _T // 2


def _sc_gather(emiss_flat, seq, base):
    info = plsc.get_sparse_core_info()
    nc, ns, lanes = info.num_cores, info.num_subcores, info.num_lanes
    nw = nc * ns
    tpw = _H // nw
    assert (tpw * _N) % 128 == 0 and (base + tpw) % 8 == 0
    rpw = tpw * _N // 128
    mesh = plsc.VectorSubcoreMesh(core_axis_name="c", subcore_axis_name="s")

    @functools.partial(
        pl.kernel,
        mesh=mesh,
        out_type=jax.ShapeDtypeStruct((_H * _N // 128, 128), jnp.float32),
        scratch_types=[
            pltpu.VMEM((tpw,), jnp.int32),
            pltpu.VMEM((tpw * _N,), jnp.int32),
            pltpu.VMEM((rpw, 128), jnp.float32),
            pltpu.SemaphoreType.DMA,
        ],
    )
    def gather_k(emiss_hbm, seq_hbm, out_hbm, seq_v, idx_v, gath_v, sem):
        wid = jax.lax.axis_index("s") * nc + jax.lax.axis_index("c")
        pltpu.sync_copy(seq_hbm.at[pl.ds(base + wid * tpw, tpw)], seq_v)
        tag = jax.lax.iota(jnp.int32, lanes) * _V
        sv = seq_v[...]
        for tl in range(tpw):
            word = sv[tl]
            for c in range(_N // lanes):
                idx_v[pl.ds(tl * _N + c * lanes, lanes)] = (
                    tag + (c * lanes * _V + word))
        copies = [
            pltpu.async_copy(
                emiss_hbm.at[idx_v.at[pl.ds(j * 128, 128)]], gath_v.at[j], sem)
            for j in range(rpw)
        ]
        for c in copies:
            c.wait()
        pltpu.sync_copy(gath_v, out_hbm.at[pl.ds(wid * rpw, rpw)])

    return gather_k(emiss_flat, seq)


def _viterbi_steps(e_ref, trans, st, mm, er_ref, ro_ref, first):
    tr_t = trans.T
    t0c_rep = jnp.broadcast_to(tr_t[:, 0:1], (_N, _N))

    def step_rc(raw_row, mm, ec_rep):
        cand = jnp.broadcast_to(raw_row, (_N, _N)) * tr_t
        m_rep = jnp.broadcast_to(
            jnp.max(cand, axis=1, keepdims=True), (_N, _N))
        sel = jnp.where(mm == 0.0, t0c_rep, m_rep)
        raw_rep = sel * ec_rep
        mm_n = jnp.max(raw_rep[:, 0:1], axis=0, keepdims=True)
        return raw_rep, mm_n

    def step_cr(raw_rep, mm, e_row):
        cand = raw_rep * trans
        m_row = jnp.max(cand, axis=0, keepdims=True)
        sel = jnp.where(mm == 0.0, trans[0:1, :], m_row)
        raw_row = sel * e_row
        mm_n = jnp.max(raw_row, axis=1, keepdims=True)
        return raw_row, mm_n

    def block16(blk_first, st, mm, e_blk):
        et_blk = e_blk.T
        rows, cols = [], []
        for j in range(16):
            if j % 2 == 0:
                if blk_first and j == 0:
                    pass
                else:
                    st, mm = step_cr(st, mm, e_blk[j:j + 1, :])
                rows.append(st)
            else:
                ec = jnp.broadcast_to(et_blk[:, j:j + 1], (_N, _N))
                st, mm = step_rc(st, mm, ec)
                cols.append(st[:, 0:1])
        even = jnp.concatenate(rows, axis=0)
        odd = jnp.concatenate(cols, axis=1).T
        return st, mm, even, odd

    lo = 0
    if first:
        e_blk0 = e_ref[0:16, :]
        st = e_blk0[0:1, :] * trans[0:1, :]
        mm = jnp.ones((1, 1), jnp.float32)
        st, mm, even, odd = block16(True, st, mm, e_blk0)
        er_ref[0:8, :] = even
        ro_ref[0:8, :] = odd
        lo = 1

    def body(kk, carry):
        st, mm = carry
        e_blk = e_ref[pl.ds(kk * 16, 16), :]
        st, mm, even, odd = block16(False, st, mm, e_blk)
        er_ref[pl.ds(kk * 8, 8), :] = even
        ro_ref[pl.ds(kk * 8, 8), :] = odd
        return st, mm

    return jax.lax.fori_loop(lo, _H // 16, body, (st, mm))


def _amax_rows(p):
    m = jnp.max(p, axis=1, keepdims=True)
    il = jax.lax.broadcasted_iota(jnp.int32, (_H // 2, _N), 1)
    return jnp.min(jnp.where(p == m, il, _N), axis=1, keepdims=True)


def _part1_body(e_ref, tr_ref, out_ref, st_ref, mm_ref, er_ref, ro_ref):
    trans = tr_ref[...]
    st, mm = _viterbi_steps(e_ref, trans, None, None, er_ref, ro_ref, True)
    st_ref[...] = st
    mm_ref[...] = mm
    out_ref[...] = jnp.concatenate(
        [_amax_rows(er_ref[...]), _amax_rows(ro_ref[...])], axis=1)


def _part2_body(e_ref, tr_ref, st_in_ref, mm_in_ref, out_ref, er_ref, ro_ref):
    trans = tr_ref[...]
    _viterbi_steps(e_ref, trans, st_in_ref[...], mm_in_ref[...],
                   er_ref, ro_ref, False)
    out_ref[...] = jnp.concatenate(
        [_amax_rows(er_ref[...]), _amax_rows(ro_ref[...])], axis=1)


_scratch = [pltpu.VMEM((_H // 2, _N), jnp.float32),
            pltpu.VMEM((_H // 2, _N), jnp.float32)]

_tc_part1 = pl.pallas_call(
    _part1_body,
    out_shape=(jax.ShapeDtypeStruct((_H // 2, 2), jnp.int32),
               jax.ShapeDtypeStruct((_N, _N), jnp.float32),
               jax.ShapeDtypeStruct((1, 1), jnp.float32)),
    scratch_shapes=_scratch,
)

_tc_part2 = pl.pallas_call(
    _part2_body,
    out_shape=jax.ShapeDtypeStruct((_H // 2, 2), jnp.int32),
    scratch_shapes=_scratch,
)


def kernel(seq, emiss, trans):
    ef = emiss.reshape(-1)
    e1 = _sc_gather(ef, seq, 0).reshape(_H, _N)
    e2 = _sc_gather(ef, seq, _H).reshape(_H, _N)
    o1, st, mm = _tc_part1(e1, trans)
    o2 = _tc_part2(e2, trans, st, mm)
    return jnp.concatenate([o1, o2], axis=0).reshape(_T)


# speedup vs baseline: 1.0410x; 1.0410x over previous
import functools

import jax
import jax.numpy as jnp
from jax.experimental import pallas as pl
from jax.experimental.pallas import tpu as pltpu
from jax.experimental.pallas import tpu_sc as plsc

_N = 64
_V = 100000
_T = 512
_OUT_TAG = 0


def _sc_gather(emiss_flat, seq):
    info = plsc.get_sparse_core_info()
    nc, ns, lanes = info.num_cores, info.num_subcores, info.num_lanes
    nw = nc * ns
    tpw = _T // nw
    assert tpw == lanes and (tpw * _N) % 128 == 0
    rpw = tpw * _N // 128
    mesh = plsc.VectorSubcoreMesh(core_axis_name="c", subcore_axis_name="s")

    @functools.partial(
        pl.kernel,
        mesh=mesh,
        out_type=jax.ShapeDtypeStruct((_T * _N // 128, 128), jnp.float32),
        scratch_types=[
            pltpu.VMEM((tpw,), jnp.int32),
            pltpu.VMEM((tpw * _N,), jnp.int32),
            pltpu.VMEM((rpw, 128), jnp.float32),
            pltpu.SemaphoreType.DMA,
        ],
    )
    def gather_k(emiss_hbm, seq_hbm, out_hbm, seq_v, idx_v, gath_v, sem):
        wid = jax.lax.axis_index("s") * nc + jax.lax.axis_index("c")
        pltpu.sync_copy(seq_hbm.at[pl.ds(wid * tpw, tpw)], seq_v)
        tag = jax.lax.iota(jnp.int32, lanes) * _V
        sv = seq_v[...]
        for tl in range(tpw):
            word = sv[tl]
            for c in range(_N // lanes):
                idx_v[pl.ds(tl * _N + c * lanes, lanes)] = (
                    tag + (c * lanes * _V + word))
        copies = [
            pltpu.async_copy(
                emiss_hbm.at[idx_v.at[pl.ds(j * 128, 128)]], gath_v.at[j], sem)
            for j in range(rpw)
        ]
        for c in copies:
            c.wait()
        pltpu.sync_copy(gath_v, out_hbm.at[pl.ds(wid * rpw, rpw)])

    return gather_k(emiss_flat, seq)


def _viterbi_body(e_ref, tr_ref, out_ref, er_ref, ro_ref):
    trans = tr_ref[...]
    tr_t = trans.T
    t0r_rep = jnp.broadcast_to(trans[0:1, :], (_N, _N))
    t0c_rep = jnp.broadcast_to(tr_t[:, 0:1], (_N, _N))

    def step_rc(raw_row, mm, ec_rep):
        cand = jnp.broadcast_to(raw_row, (_N, _N)) * tr_t
        m_rep = jnp.broadcast_to(
            jnp.max(cand, axis=1, keepdims=True), (_N, _N))
        sel = jnp.where(mm == 0.0, t0c_rep, m_rep)
        raw_rep = sel * ec_rep
        mm_n = jnp.max(raw_rep[:, 0:1], axis=0, keepdims=True)
        return raw_rep, mm_n

    def step_cr(raw_rep, mm, e_row):
        cand = raw_rep * trans
        m_row = jnp.max(cand, axis=0, keepdims=True)
        sel = jnp.where(mm == 0.0, trans[0:1, :], m_row)
        raw_row = sel * e_row
        mm_n = jnp.max(raw_row, axis=1, keepdims=True)
        return raw_row, mm_n

    def block16(first, st, mm, e_blk):
        et_blk = e_blk.T
        rows, cols = [], []
        for j in range(16):
            if j % 2 == 0:
                if first and j == 0:
                    pass
                else:
                    st, mm = step_cr(st, mm, e_blk[j:j + 1, :])
                rows.append(st)
            else:
                ec = jnp.broadcast_to(et_blk[:, j:j + 1], (_N, _N))
                st, mm = step_rc(st, mm, ec)
                cols.append(st[:, 0:1])
        even = jnp.concatenate(rows, axis=0)
        odd = jnp.concatenate(cols, axis=1).T
        return st, mm, even, odd

    e_blk0 = e_ref[0:16, :]
    st = e_blk0[0:1, :] * trans[0:1, :]
    mm = jnp.ones((1, 1), jnp.float32)
    st, mm, even, odd = block16(True, st, mm, e_blk0)
    er_ref[0:8, :] = even
    ro_ref[0:8, :] = odd

    def body(kk, carry):
        st, mm = carry
        e_blk = e_ref[pl.ds(kk * 16, 16), :]
        st, mm, even, odd = block16(False, st, mm, e_blk)
        er_ref[pl.ds(kk * 8, 8), :] = even
        ro_ref[pl.ds(kk * 8, 8), :] = odd
        return st, mm

    jax.lax.fori_loop(1, _T // 16, body, (st, mm))

    def amax_rows(p):
        m = jnp.max(p, axis=1, keepdims=True)
        il = jax.lax.broadcasted_iota(jnp.int32, (_T // 2, _N), 1)
        return jnp.min(jnp.where(p == m, il, _N), axis=1, keepdims=True)

    ae = amax_rows(er_ref[...])
    ao = amax_rows(ro_ref[...])
    out_ref[...] = jnp.concatenate([ae, ao], axis=1)


_tc_viterbi = pl.pallas_call(
    _viterbi_body,
    out_shape=jax.ShapeDtypeStruct((_T // 2, 2), jnp.int32),
    scratch_shapes=[pltpu.VMEM((_T // 2, _N), jnp.float32),
                    pltpu.VMEM((_T // 2, _N), jnp.float32)],
)


def kernel(seq, emiss, trans):
    e256 = _sc_gather(emiss.reshape(-1), seq)
    e_rows = e256.reshape(_T, _N)
    return _tc_viterbi(e_rows, trans).reshape(_T)


# speedup vs baseline: 1.0413x; 1.0002x over previous
import functools

import jax
import jax.numpy as jnp
from jax.experimental import pallas as pl
from jax.experimental.pallas import tpu as pltpu
from jax.experimental.pallas import tpu_sc as plsc

_N = 64
_V = 100000
_T = 512


def _sc_gather(emiss_flat, seq):
    info = plsc.get_sparse_core_info()
    nc, ns, lanes = info.num_cores, info.num_subcores, info.num_lanes
    nw = nc * ns
    tpw = _T // nw
    assert tpw == lanes and (tpw * _N) % 128 == 0
    rpw = tpw * _N // 128
    mesh = plsc.VectorSubcoreMesh(core_axis_name="c", subcore_axis_name="s")

    @functools.partial(
        pl.kernel,
        mesh=mesh,
        out_type=jax.ShapeDtypeStruct((_T * _N // 128, 128), jnp.float32),
        scratch_types=[
            pltpu.VMEM((tpw,), jnp.int32),
            pltpu.VMEM((tpw * _N,), jnp.int32),
            pltpu.VMEM((rpw, 128), jnp.float32),
            pltpu.SemaphoreType.DMA,
        ],
    )
    def gather_k(emiss_hbm, seq_hbm, out_hbm, seq_v, idx_v, gath_v, sem):
        wid = jax.lax.axis_index("s") * nc + jax.lax.axis_index("c")
        pltpu.sync_copy(seq_hbm.at[pl.ds(wid * tpw, tpw)], seq_v)
        tag = jax.lax.iota(jnp.int32, lanes) * _V
        sv = seq_v[...]
        for tl in range(tpw):
            word = sv[tl]
            for c in range(_N // lanes):
                idx_v[pl.ds(tl * _N + c * lanes, lanes)] = (
                    tag + (c * lanes * _V + word))
        copies = [
            pltpu.async_copy(
                emiss_hbm.at[idx_v.at[pl.ds(j * 128, 128)]], gath_v.at[j], sem)
            for j in range(rpw)
        ]
        for c in copies:
            c.wait()
        pltpu.sync_copy(gath_v, out_hbm.at[pl.ds(wid * rpw, rpw)])

    return gather_k(emiss_flat, seq)


def _viterbi_body(e_ref, tr_ref, out_ref, er_ref, ro_ref):
    trans = tr_ref[...]
    tr_t = trans.T
    t0c_rep = jnp.broadcast_to(tr_t[:, 0:1], (_N, _N))

    def step_rc(raw_row, mm, ec_rep):
        cand = jnp.broadcast_to(raw_row, (_N, _N)) * tr_t
        m_rep = jnp.broadcast_to(
            jnp.max(cand, axis=1, keepdims=True), (_N, _N))
        sel = jnp.where(mm == 0.0, t0c_rep, m_rep)
        raw_rep = sel * ec_rep
        mm_n = jnp.max(raw_rep[:, 0:1], axis=0, keepdims=True)
        return raw_rep, mm_n

    def step_cr(raw_rep, mm, e_row):
        cand = raw_rep * trans
        m_row = jnp.max(cand, axis=0, keepdims=True)
        sel = jnp.where(mm == 0.0, trans[0:1, :], m_row)
        raw_row = sel * e_row
        mm_n = jnp.max(raw_row, axis=1, keepdims=True)
        return raw_row, mm_n

    def block16(first, st, mm, e_blk):
        et_blk = e_blk.T
        rows, cols = [], []
        for j in range(16):
            if j % 2 == 0:
                if first and j == 0:
                    pass
                else:
                    st, mm = step_cr(st, mm, e_blk[j:j + 1, :])
                rows.append(st)
            else:
                ec = jnp.broadcast_to(et_blk[:, j:j + 1], (_N, _N))
                st, mm = step_rc(st, mm, ec)
                cols.append(st[:, 0:1])
        even = jnp.concatenate(rows, axis=0)
        odd = jnp.concatenate(cols, axis=1).T
        return st, mm, even, odd

    e_blk0 = e_ref[0:16, :]
    st = e_blk0[0:1, :] * trans[0:1, :]
    mm = jnp.ones((1, 1), jnp.float32)
    st, mm, even, odd = block16(True, st, mm, e_blk0)
    er_ref[0:8, :] = even
    ro_ref[0:8, :] = odd

    def body(kk, carry):
        st, mm = carry
        e_blk = e_ref[pl.ds(kk * 16, 16), :]
        st, mm, even, odd = block16(False, st, mm, e_blk)
        er_ref[pl.ds(kk * 8, 8), :] = even
        ro_ref[pl.ds(kk * 8, 8), :] = odd
        return st, mm

    jax.lax.fori_loop(1, _T // 16, body, (st, mm))

    def amax_rows(p):
        m = jnp.max(p, axis=1, keepdims=True)
        il = jax.lax.broadcasted_iota(jnp.int32, (_T // 2, _N), 1)
        return jnp.min(jnp.where(p == m, il, _N), axis=1, keepdims=True)

    ae = amax_rows(er_ref[...])
    ao = amax_rows(ro_ref[...])
    out_ref[...] = jnp.concatenate([ae, ao], axis=1)


_tc_viterbi = pl.pallas_call(
    _viterbi_body,
    out_shape=jax.ShapeDtypeStruct((_T // 2, 2), jnp.int32),
    scratch_shapes=[pltpu.VMEM((_T // 2, _N), jnp.float32),
                    pltpu.VMEM((_T // 2, _N), jnp.float32)],
)


def kernel(seq, emiss, trans):
    e256 = _sc_gather(emiss.reshape(-1), seq)
    e_rows = e256.reshape(_T, _N)
    return _tc_viterbi(e_rows, trans).reshape(_T)
